# baseline (speedup 1.0000x reference)
import functools
import jax
import jax.numpy as jnp
from jax import lax
from jax.experimental import pallas as pl
from jax.experimental.pallas import tpu as pltpu
from jax.experimental.pallas import tpu_sc as plsc

N = 10000
E = 320000
CH = 128
KAPPA = 0.95

NC, NS = 2, 16
NW = NC * NS
C = 80
NCHUNK = (E // NW) // C
EPAD = NW * NCHUNK * C
NP = 10112
RPS = NP // NS

BR = 2000


def _tc1_body(x_ref, W1_ref, u1_ref):
    u1_ref[...] = jnp.dot(x_ref[...].astype(jnp.bfloat16),
                          W1_ref[...].astype(jnp.bfloat16),
                          preferred_element_type=jnp.float32)


_tc1 = pl.pallas_call(
    _tc1_body,
    grid=(N // BR,),
    in_specs=[
        pl.BlockSpec((BR, CH), lambda i: (i, 0)),
        pl.BlockSpec((CH, CH), lambda i: (0, 0)),
    ],
    out_specs=pl.BlockSpec((BR, CH), lambda i: (i, 0)),
    out_shape=jax.ShapeDtypeStruct((N, CH), jnp.float32),
)


def _tcw_body(emb_ref, F_ref, e2_ref, wp_ref):
    @pl.when(pl.program_id(0) == 0)
    def _():
        Fm = F_ref[...]
        Wp = lax.dot_general(Fm, Fm, (((0,), (0,)), ((), ())),
                             preferred_element_type=jnp.float32)
        nrm = jnp.sqrt(jnp.sum(Wp * Wp))
        Wp = jnp.where(nrm > 1.0, Wp / (nrm + 1e-5), Wp)
        wp_ref[...] = Wp * KAPPA
    e2_ref[...] = jnp.dot(emb_ref[...], wp_ref[...],
                          preferred_element_type=jnp.float32)


_tcw = pl.pallas_call(
    _tcw_body,
    grid=(N // BR,),
    in_specs=[
        pl.BlockSpec((BR, CH), lambda i: (i, 0)),
        pl.BlockSpec((CH, CH), lambda i: (0, 0)),
    ],
    out_specs=pl.BlockSpec((BR, CH), lambda i: (i, 0)),
    out_shape=jax.ShapeDtypeStruct((N, CH), jnp.float32),
    scratch_shapes=[pltpu.VMEM((CH, CH), jnp.float32)],
)


def _tc2_body(p0_ref, p1_ref, b1_ref, W2_ref, e2_ref, u2_ref):
    h = jnp.maximum(p0_ref[0] + p1_ref[0] + b1_ref[...], 0.0)
    u2_ref[...] = jnp.dot(h.astype(jnp.bfloat16),
                          W2_ref[...].astype(jnp.bfloat16),
                          preferred_element_type=jnp.float32) + e2_ref[...]


_tc2 = pl.pallas_call(
    _tc2_body,
    grid=(N // BR,),
    in_specs=[
        pl.BlockSpec((1, BR, CH), lambda i: (0, i, 0)),
        pl.BlockSpec((1, BR, CH), lambda i: (1, i, 0)),
        pl.BlockSpec((1, CH), lambda i: (0, 0)),
        pl.BlockSpec((CH, CH), lambda i: (0, 0)),
        pl.BlockSpec((BR, CH), lambda i: (i, 0)),
    ],
    out_specs=pl.BlockSpec((BR, CH), lambda i: (i, 0)),
    out_shape=jax.ShapeDtypeStruct((N, CH), jnp.float32),
)


def _tc3_body(q0_ref, q1_ref, b2_ref, out_ref):
    out_ref[...] = q0_ref[0] + q1_ref[0] + b2_ref[...]


_tc3 = pl.pallas_call(
    _tc3_body,
    grid=(N // BR,),
    in_specs=[
        pl.BlockSpec((1, BR, CH), lambda i: (0, i, 0)),
        pl.BlockSpec((1, BR, CH), lambda i: (1, i, 0)),
        pl.BlockSpec((1, CH), lambda i: (0, 0)),
    ],
    out_specs=pl.BlockSpec((BR, CH), lambda i: (i, 0)),
    out_shape=jax.ShapeDtypeStruct((N, CH), jnp.float32),
)


def _pack_body(ei_ref, pk_ref):
    pk_ref[...] = jnp.bitwise_or(ei_ref[0], jnp.left_shift(ei_ref[1], 16))


_pack = pl.pallas_call(
    _pack_body,
    out_shape=jax.ShapeDtypeStruct((E,), jnp.int32),
)


_sc_mesh = plsc.VectorSubcoreMesh(
    core_axis_name="c", subcore_axis_name="s", num_cores=NC, num_subcores=NS)


@functools.partial(
    pl.kernel,
    out_type=jax.ShapeDtypeStruct((NC, NP, CH), jnp.float32),
    mesh=_sc_mesh,
    scratch_types=[
        pltpu.VMEM((NCHUNK * C,), jnp.int32),
        pltpu.VMEM((2, C), jnp.int32),
        pltpu.VMEM((2, C), jnp.int32),
        pltpu.VMEM((2, C), jnp.int32),
        pltpu.VMEM((C, CH), jnp.float32),
        pltpu.VMEM((C, CH), jnp.float32),
        pltpu.VMEM((C, CH), jnp.float32),
        pltpu.VMEM_SHARED((NP, CH), jnp.float32),
        pltpu.SemaphoreType.DMA,
        pltpu.SemaphoreType.DMA,
        pltpu.SemaphoreType.DMA,
        pltpu.SemaphoreType.DMA,
        pltpu.SemaphoreType.DMA,
        pltpu.SemaphoreType.DMA,
    ],
)
def _segsum(u_hbm, pk_hbm, out_hbm,
            pk_v, idx0, idx1, idx2, buf0, buf1, buf2, acc,
            gs0, gs1, gs2, ss0, ss1, ss2):
    idxs = (idx0, idx1, idx2)
    bufs = (buf0, buf1, buf2)
    gsems = (gs0, gs1, gs2)
    ssems = (ss0, ss1, ss2)
    cid = lax.axis_index("c")
    sid = lax.axis_index("s")
    wid = sid * NC + cid
    pltpu.sync_copy(pk_hbm.at[pl.ds(wid * (NCHUNK * C), NCHUNK * C)], pk_v)

    def unpack(j, idx_c):
        for k in range(C // 16):
            p = pk_v[pl.ds(j * C + k * 16, 16)]
            idx_c[0, pl.ds(k * 16, 16)] = lax.bitwise_and(p, 0xFFFF)
            idx_c[1, pl.ds(k * 16, 16)] = lax.shift_right_logical(p, 16)

    def wait_gather(b):
        pltpu.make_async_copy(u_hbm.at[idxs[b].at[0]], bufs[b], gsems[b]).wait()

    def fire_gather(j, b):
        unpack(j, idxs[b])
        pltpu.async_copy(u_hbm.at[idxs[b].at[0]], bufs[b], gsems[b])

    def fire_scatter(b):
        pltpu.async_copy(bufs[b], acc.at[idxs[b].at[1]], ssems[b], add=True)

    def wait_scatter(b):
        pltpu.make_async_copy(bufs[b], acc.at[idxs[b].at[1]], ssems[b]).wait()

    fire_gather(0, 0)
    fire_gather(1, 1)

    zv = jnp.zeros((16,), jnp.float32)

    def zrow(r, carry):
        for kk in range(CH // 16):
            buf2[r, pl.ds(kk * 16, 16)] = zv
        return carry

    lax.fori_loop(0, C, zrow, 0)
    zbase = sid * RPS
    ztail = RPS - 7 * C
    for t in range(7):
        pltpu.async_copy(buf2, acc.at[pl.ds(zbase + t * C, C)], gs2)
    pltpu.async_copy(buf2.at[pl.ds(0, ztail)],
                     acc.at[pl.ds(zbase + 7 * C, ztail)], gs2)
    for t in range(7):
        pltpu.make_async_copy(buf2, acc.at[pl.ds(zbase + t * C, C)], gs2).wait()
    pltpu.make_async_copy(buf2.at[pl.ds(0, ztail)],
                          acc.at[pl.ds(zbase + 7 * C, ztail)], gs2).wait()
    fire_gather(2, 2)
    plsc.subcore_barrier()

    def group(i, carry):
        for b3 in range(3):
            k = 3 * i + b3
            b2 = (b3 + 2) % 3

            @pl.when(k < NCHUNK)
            def _():
                wait_gather(b3)
                fire_scatter(b3)

            @pl.when((k >= 1) & (k + 2 < NCHUNK))
            def _():
                wait_scatter(b2)
                fire_gather(k + 2, b2)

        return carry

    lax.fori_loop(0, (NCHUNK + 2) // 3, group, 0)
    for b in range(3):
        wait_scatter(b)
    plsc.subcore_barrier()
    pltpu.sync_copy(acc.at[pl.ds(sid * RPS, RPS)],
                    out_hbm.at[cid, pl.ds(sid * RPS, RPS)])


def kernel(x, edge_index, W1, b1, W2, b2, F, emb):
    packed = _pack(edge_index)
    u1 = _tc1(x, W1)
    p = _segsum(u1, packed)
    e2 = _tcw(emb, F)
    u2 = _tc2(p, p, b1.reshape(1, CH), W2, e2)
    q = _segsum(u2, packed)
    return _tc3(q, q, b2.reshape(1, CH))


# speedup vs baseline: 1.0021x; 1.0021x over previous
import functools
import jax
import jax.numpy as jnp
from jax import lax
from jax.experimental import pallas as pl
from jax.experimental.pallas import tpu as pltpu
from jax.experimental.pallas import tpu_sc as plsc

N = 10000
E = 320000
CH = 128
KAPPA = 0.95

NC, NS = 2, 16
NW = NC * NS
C = 80
NCHUNK = (E // NW) // C
EPAD = NW * NCHUNK * C
NP = 10112
RPS = NP // NS

BR = 2000


def _tc1_body(x_ref, W1_ref, u1_ref):
    u1_ref[...] = jnp.dot(x_ref[...], W1_ref[...],
                          preferred_element_type=jnp.float32)


_tc1 = pl.pallas_call(
    _tc1_body,
    grid=(N // BR,),
    in_specs=[
        pl.BlockSpec((BR, CH), lambda i: (i, 0)),
        pl.BlockSpec((CH, CH), lambda i: (0, 0)),
    ],
    out_specs=pl.BlockSpec((BR, CH), lambda i: (i, 0)),
    out_shape=jax.ShapeDtypeStruct((N, CH), jnp.float32),
)


def _pack_body(ei_ref, pk_ref):
    pk_ref[...] = jnp.bitwise_or(ei_ref[0], jnp.left_shift(ei_ref[1], 16))


_pack = pl.pallas_call(
    _pack_body,
    out_shape=jax.ShapeDtypeStruct((E,), jnp.int32),
)


def _tcw_body(emb_ref, F_ref, e2_ref, wp_ref):
    @pl.when(pl.program_id(0) == 0)
    def _():
        Fm = F_ref[...]
        Wp = lax.dot_general(Fm, Fm, (((0,), (0,)), ((), ())),
                             preferred_element_type=jnp.float32)
        nrm = jnp.sqrt(jnp.sum(Wp * Wp))
        Wp = jnp.where(nrm > 1.0, Wp / (nrm + 1e-5), Wp)
        wp_ref[...] = Wp * KAPPA
    e2_ref[...] = jnp.dot(emb_ref[...], wp_ref[...],
                          preferred_element_type=jnp.float32)


_tcw = pl.pallas_call(
    _tcw_body,
    grid=(N // BR,),
    in_specs=[
        pl.BlockSpec((BR, CH), lambda i: (i, 0)),
        pl.BlockSpec((CH, CH), lambda i: (0, 0)),
    ],
    out_specs=pl.BlockSpec((BR, CH), lambda i: (i, 0)),
    out_shape=jax.ShapeDtypeStruct((N, CH), jnp.float32),
    scratch_shapes=[pltpu.VMEM((CH, CH), jnp.float32)],
)


def _tc2_body(p0_ref, p1_ref, b1_ref, W2_ref, e2_ref, u2_ref):
    h = jnp.maximum(p0_ref[0] + p1_ref[0] + b1_ref[...], 0.0)
    u2_ref[...] = jnp.dot(h, W2_ref[...],
                          preferred_element_type=jnp.float32) + e2_ref[...]


_tc2 = pl.pallas_call(
    _tc2_body,
    grid=(N // BR,),
    in_specs=[
        pl.BlockSpec((1, BR, CH), lambda i: (0, i, 0)),
        pl.BlockSpec((1, BR, CH), lambda i: (1, i, 0)),
        pl.BlockSpec((1, CH), lambda i: (0, 0)),
        pl.BlockSpec((CH, CH), lambda i: (0, 0)),
        pl.BlockSpec((BR, CH), lambda i: (i, 0)),
    ],
    out_specs=pl.BlockSpec((BR, CH), lambda i: (i, 0)),
    out_shape=jax.ShapeDtypeStruct((N, CH), jnp.float32),
)


def _tc3_body(q0_ref, q1_ref, b2_ref, out_ref):
    out_ref[...] = q0_ref[0] + q1_ref[0] + b2_ref[...]


_tc3 = pl.pallas_call(
    _tc3_body,
    grid=(N // BR,),
    in_specs=[
        pl.BlockSpec((1, BR, CH), lambda i: (0, i, 0)),
        pl.BlockSpec((1, BR, CH), lambda i: (1, i, 0)),
        pl.BlockSpec((1, CH), lambda i: (0, 0)),
    ],
    out_specs=pl.BlockSpec((BR, CH), lambda i: (i, 0)),
    out_shape=jax.ShapeDtypeStruct((N, CH), jnp.float32),
)


_sc_mesh = plsc.VectorSubcoreMesh(
    core_axis_name="c", subcore_axis_name="s", num_cores=NC, num_subcores=NS)


@functools.partial(
    pl.kernel,
    out_type=jax.ShapeDtypeStruct((NC, NP, CH), jnp.float32),
    mesh=_sc_mesh,
    scratch_types=[
        pltpu.VMEM((NCHUNK * C,), jnp.int32),
        pltpu.VMEM((2, C), jnp.int32),
        pltpu.VMEM((2, C), jnp.int32),
        pltpu.VMEM((2, C), jnp.int32),
        pltpu.VMEM((C, CH), jnp.float32),
        pltpu.VMEM((C, CH), jnp.float32),
        pltpu.VMEM((C, CH), jnp.float32),
        pltpu.VMEM_SHARED((NP, CH), jnp.float32),
        pltpu.SemaphoreType.DMA,
        pltpu.SemaphoreType.DMA,
        pltpu.SemaphoreType.DMA,
        pltpu.SemaphoreType.DMA,
        pltpu.SemaphoreType.DMA,
        pltpu.SemaphoreType.DMA,
    ],
)
def _segsum(u_hbm, pk_hbm, out_hbm,
            pk_v, idx0, idx1, idx2, buf0, buf1, buf2, acc,
            gs0, gs1, gs2, ss0, ss1, ss2):
    idxs = (idx0, idx1, idx2)
    bufs = (buf0, buf1, buf2)
    gsems = (gs0, gs1, gs2)
    ssems = (ss0, ss1, ss2)
    cid = lax.axis_index("c")
    sid = lax.axis_index("s")
    wid = sid * NC + cid
    pltpu.sync_copy(pk_hbm.at[pl.ds(wid * (NCHUNK * C), NCHUNK * C)], pk_v)

    def unpack(j, idx_c):
        for k in range(C // 16):
            p = pk_v[pl.ds(j * C + k * 16, 16)]
            idx_c[0, pl.ds(k * 16, 16)] = lax.bitwise_and(p, 0xFFFF)
            idx_c[1, pl.ds(k * 16, 16)] = lax.shift_right_logical(p, 16)

    def wait_gather(b):
        pltpu.make_async_copy(u_hbm.at[idxs[b].at[0]], bufs[b], gsems[b]).wait()

    def fire_gather(j, b):
        unpack(j, idxs[b])
        pltpu.async_copy(u_hbm.at[idxs[b].at[0]], bufs[b], gsems[b])

    def fire_scatter(b):
        pltpu.async_copy(bufs[b], acc.at[idxs[b].at[1]], ssems[b], add=True)

    def wait_scatter(b):
        pltpu.make_async_copy(bufs[b], acc.at[idxs[b].at[1]], ssems[b]).wait()

    fire_gather(0, 0)
    fire_gather(1, 1)

    zv = jnp.zeros((16,), jnp.float32)

    def zrow(r, carry):
        for kk in range(CH // 16):
            buf2[r, pl.ds(kk * 16, 16)] = zv
        return carry

    lax.fori_loop(0, C, zrow, 0)
    zbase = sid * RPS
    ztail = RPS - 7 * C
    for t in range(7):
        pltpu.async_copy(buf2, acc.at[pl.ds(zbase + t * C, C)], gs2)
    pltpu.async_copy(buf2.at[pl.ds(0, ztail)],
                     acc.at[pl.ds(zbase + 7 * C, ztail)], gs2)
    for t in range(7):
        pltpu.make_async_copy(buf2, acc.at[pl.ds(zbase + t * C, C)], gs2).wait()
    pltpu.make_async_copy(buf2.at[pl.ds(0, ztail)],
                          acc.at[pl.ds(zbase + 7 * C, ztail)], gs2).wait()
    fire_gather(2, 2)
    plsc.subcore_barrier()

    def group(i, carry):
        for b3 in range(3):
            k = 3 * i + b3
            b2 = (b3 + 2) % 3

            @pl.when(k < NCHUNK)
            def _():
                wait_gather(b3)
                fire_scatter(b3)

            @pl.when((k >= 1) & (k + 2 < NCHUNK))
            def _():
                wait_scatter(b2)
                fire_gather(k + 2, b2)

        return carry

    lax.fori_loop(0, (NCHUNK + 2) // 3, group, 0)
    for b in range(3):
        wait_scatter(b)
    plsc.subcore_barrier()
    pltpu.sync_copy(acc.at[pl.ds(sid * RPS, RPS)],
                    out_hbm.at[cid, pl.ds(sid * RPS, RPS)])


def kernel(x, edge_index, W1, b1, W2, b2, F, emb):
    packed = _pack(edge_index)
    u1 = _tc1(x, W1)
    p = _segsum(u1, packed)
    e2 = _tcw(emb, F)
    u2 = _tc2(p, p, b1.reshape(1, CH), W2, e2)
    q = _segsum(u2, packed)
    return _tc3(q, q, b2.reshape(1, CH))
